# baseline (speedup 1.0000x reference)
import functools

import numpy as np
import jax
import jax.numpy as jnp
from jax import lax
from jax.experimental import pallas as pl
from jax.experimental.pallas import tpu as pltpu
from jax.experimental.pallas import tpu_sc as plsc

N_AC = 20000
BATCH = 8
_rng = np.random.RandomState(42)
_ctr = _rng.uniform(0.0, 1.0, size=(N_AC, 2))
_wh = _rng.uniform(0.02, 0.3, size=(N_AC, 2))
_anchors = np.concatenate([_ctr - _wh / 2.0, _ctr + _wh / 2.0], axis=1).astype(np.float32)
_mask = np.all((_anchors >= 0.0) & (_anchors <= 1.0), axis=1)
_idx = np.nonzero(_mask)[0].astype(np.int32)
N_VAL = int(_idx.shape[0])

NQ = 4
N_PAD = 14080
CHUNK = N_PAD // NQ
GROUPS = CHUNK // 16

_idx_pad = np.concatenate(
    [_idx, np.full((N_PAD - N_VAL,), _idx[-1], np.int32)])
_los = []
_spans = []
for _q in range(NQ):
    _first = int(_idx_pad[_q * CHUNK])
    _last = int(_idx_pad[(_q + 1) * CHUNK - 1])
    _lo = _first & ~7
    _los.append(_lo)
    _spans.append(_last - _lo + 1)
L_IN = -(-max(_spans) // 8) * 8
_los = [min(_lo, N_AC - L_IN) for _lo in _los]
_LDX_np = np.stack(
    [_idx_pad[_q * CHUNK:(_q + 1) * CHUNK] - _los[_q] for _q in range(NQ)]
).astype(np.int32)

_va = _anchors[_mask]
_aw = _va[:, 2] - _va[:, 0]
_ah = _va[:, 3] - _va[:, 1]
_acx = _va[:, 0] + 0.5 * _aw
_acy = _va[:, 1] + 0.5 * _ah
_comp = np.stack([_aw, _ah, _acx, _acy])
_comp = np.concatenate(
    [_comp, np.repeat(_comp[:, -1:], N_PAD - N_VAL, axis=1)], axis=1)
_CST_np = np.ascontiguousarray(
    _comp.reshape(4, NQ, CHUNK).transpose(1, 0, 2)).astype(np.float32)

_LDX = jnp.asarray(_LDX_np.reshape(-1))
_CST = jnp.asarray(_CST_np.reshape(-1))

MAXR = float(np.abs(np.log(1000.0 / 16.0)))
NC, NS = 2, 16


@functools.lru_cache(maxsize=None)
def _build():
    mesh = plsc.VectorSubcoreMesh(
        core_axis_name="c", subcore_axis_name="s",
        num_cores=NC, num_subcores=NS)

    @functools.partial(
        pl.kernel,
        mesh=mesh,
        compiler_params=pltpu.CompilerParams(
            use_tc_tiling_on_sc=False, needs_layout_passes=False),
        out_type=[
            jax.ShapeDtypeStruct((BATCH * N_PAD,), jnp.float32),
            jax.ShapeDtypeStruct((BATCH * N_PAD * 4,), jnp.float32),
            jax.ShapeDtypeStruct((BATCH * N_PAD * 4,), jnp.float32),
        ],
        scratch_types=[
            pltpu.VMEM((L_IN, 1), jnp.float32),
            pltpu.VMEM((L_IN, 4), jnp.float32),
            pltpu.VMEM((CHUNK,), jnp.int32),
            pltpu.VMEM((4 * CHUNK,), jnp.float32),
            pltpu.VMEM((CHUNK,), jnp.float32),
            pltpu.VMEM((CHUNK * 4,), jnp.float32),
            pltpu.VMEM((CHUNK * 4,), jnp.float32),
        ],
    )
    def _roi_sc(cls_hbm, del_hbm, ldx_hbm, cst_hbm,
                cls_out, del_out, roi_out,
                cls_v, del_v, ldx_v, cst_v, clsb, delb, roib):
        wid = lax.axis_index("s") * NC + lax.axis_index("c")
        b = wid // NQ
        q = wid % NQ
        lo = jnp.int32(_los[0])
        for qq in range(1, NQ):
            lo = jnp.where(q == qq, jnp.int32(_los[qq]), lo)

        lo = pl.multiple_of(lo, 8)
        ol = pl.multiple_of(q * CHUNK, 8)
        ok = pl.multiple_of(q * (4 * CHUNK), 8)
        pltpu.sync_copy(cls_hbm.at[b, pl.ds(lo, L_IN), :], cls_v)
        pltpu.sync_copy(del_hbm.at[b, pl.ds(lo, L_IN), :], del_v)
        pltpu.sync_copy(ldx_hbm.at[pl.ds(ol, CHUNK)], ldx_v)
        pltpu.sync_copy(cst_hbm.at[pl.ds(ok, 4 * CHUNK)], cst_v)

        def body(g, carry):
            base = g * 16
            rows = ldx_v[pl.ds(base, 16)]
            zero = jnp.full((16,), 0, jnp.int32)
            orow4 = (lax.iota(jnp.int32, 16) + base) * 4
            clsb[pl.ds(base, 16)] = plsc.load_gather(cls_v, [rows, zero])
            d = []
            for c in range(4):
                col = jnp.full((16,), c, jnp.int32)
                dc = plsc.load_gather(del_v, [rows, col])
                plsc.store_scatter(delb, [orow4 + c], dc)
                d.append(dc)
            aw = cst_v[pl.ds(base, 16)]
            ah = cst_v[pl.ds(CHUNK + base, 16)]
            acx = cst_v[pl.ds(2 * CHUNK + base, 16)]
            acy = cst_v[pl.ds(3 * CHUNK + base, 16)]
            dw = jnp.clip(d[2], -MAXR, MAXR)
            dh = jnp.clip(d[3], -MAXR, MAXR)
            pcx = acx + d[0] * aw
            pcy = acy + d[1] * ah
            pw = 0.5 * aw * jnp.exp(dw)
            ph = 0.5 * ah * jnp.exp(dh)
            r = [pcx - pw, pcy - ph, pcx + pw, pcy + ph]
            for c in range(4):
                plsc.store_scatter(
                    roib, [orow4 + c], jnp.clip(r[c], -0.1, 1.1))
            return carry

        lax.fori_loop(0, GROUPS, body, 0)

        osc = pl.multiple_of(b * N_PAD + q * CHUNK, 8)
        osd = pl.multiple_of((b * N_PAD + q * CHUNK) * 4, 8)
        pltpu.sync_copy(clsb, cls_out.at[pl.ds(osc, CHUNK)])
        pltpu.sync_copy(delb, del_out.at[pl.ds(osd, CHUNK * 4)])
        pltpu.sync_copy(roib, roi_out.at[pl.ds(osd, CHUNK * 4)])

    return _roi_sc


@jax.jit
def kernel(rpn_cls, rpn_del):
    cls_o, del_o, roi_o = _build()(rpn_cls, rpn_del, _LDX, _CST)
    cls_o = cls_o.reshape(BATCH, N_PAD)[:, :N_VAL, None]
    del_o = del_o.reshape(BATCH, N_PAD, 4)[:, :N_VAL]
    roi_o = roi_o.reshape(BATCH, N_PAD, 4)[:, :N_VAL]
    return (cls_o, del_o, roi_o)


# speedup vs baseline: 17.1089x; 17.1089x over previous
import functools

import numpy as np
import jax
import jax.numpy as jnp
from jax import lax
from jax.experimental import pallas as pl
from jax.experimental.pallas import tpu as pltpu
from jax.experimental.pallas import tpu_sc as plsc

N_AC = 20000
BATCH = 8
_rng = np.random.RandomState(42)
_ctr = _rng.uniform(0.0, 1.0, size=(N_AC, 2))
_wh = _rng.uniform(0.02, 0.3, size=(N_AC, 2))
_anchors = np.concatenate([_ctr - _wh / 2.0, _ctr + _wh / 2.0], axis=1).astype(np.float32)
_mask = np.all((_anchors >= 0.0) & (_anchors <= 1.0), axis=1)
_idx = np.nonzero(_mask)[0].astype(np.int32)
N_VAL = int(_idx.shape[0])

NQ = 4
CHUNK = 3584
LENS = [3584, 3584, 3584, 3328]
GROUPS = CHUNK // 16

_idx_pad = np.concatenate(
    [_idx, np.full((NQ * CHUNK - N_VAL,), _idx[-1], np.int32)])
_los = []
_spans = []
for _q in range(NQ):
    _first = int(_idx_pad[_q * CHUNK])
    _last = int(_idx_pad[(_q + 1) * CHUNK - 1])
    _lo = _first & ~127
    _los.append(_lo)
    _spans.append(_last - _lo + 1)
L_IN = -(-max(_spans) // 128) * 128
_LDX_np = np.stack(
    [_idx_pad[_q * CHUNK:(_q + 1) * CHUNK] - _los[_q] for _q in range(NQ)]
).astype(np.int32)

_va = _anchors[_mask]
_aw = _va[:, 2] - _va[:, 0]
_ah = _va[:, 3] - _va[:, 1]
_acx = _va[:, 0] + 0.5 * _aw
_acy = _va[:, 1] + 0.5 * _ah
_comp = np.stack([_aw, _ah, _acx, _acy])
_comp = np.concatenate(
    [_comp, np.repeat(_comp[:, -1:], NQ * CHUNK - N_VAL, axis=1)], axis=1)
_CST_np = np.ascontiguousarray(
    _comp.reshape(4, NQ, CHUNK).transpose(1, 0, 2)).astype(np.float32)

_LDX = jnp.asarray(_LDX_np.reshape(-1))
_CST = jnp.asarray(_CST_np.reshape(-1))

MAXR = float(np.abs(np.log(1000.0 / 16.0)))
NC, NS = 2, 16


@functools.lru_cache(maxsize=None)
def _build():
    mesh = plsc.VectorSubcoreMesh(
        core_axis_name="c", subcore_axis_name="s",
        num_cores=NC, num_subcores=NS)

    @functools.partial(
        pl.kernel,
        mesh=mesh,
        compiler_params=pltpu.CompilerParams(
            use_tc_tiling_on_sc=True, needs_layout_passes=False,
            disable_bounds_checks=True),
        out_type=[
            jax.ShapeDtypeStruct((BATCH, 1, N_VAL), jnp.float32),
            jax.ShapeDtypeStruct((BATCH, 4, N_VAL), jnp.float32),
            jax.ShapeDtypeStruct((BATCH, 4, N_VAL), jnp.float32),
        ],
        scratch_types=[
            pltpu.VMEM((NQ * CHUNK,), jnp.int32),
        ],
    )
    def _roi_sc(cls_hbm, del_hbm, ldx_hbm, cst_hbm,
                cls_out, del_out, roi_out, ldx_v):
        wid = lax.axis_index("s") * NC + lax.axis_index("c")
        b = wid // NQ
        q = wid % NQ
        lo = jnp.int32(_los[0])
        for qq in range(1, NQ):
            lo = jnp.where(q == qq, jnp.int32(_los[qq]), lo)
        lo = pl.multiple_of(lo, 128)
        ol = pl.multiple_of(q * CHUNK, 8)
        ost = pl.multiple_of(q * CHUNK, 128)

        pltpu.sync_copy(ldx_hbm.at[pl.ds(ol, CHUNK)], ldx_v.at[pl.ds(0, CHUNK)])

        def del_phase(del_v, cst_v, delb, roib):
            ok = pl.multiple_of(q * (4 * CHUNK), 8)
            pltpu.sync_copy(del_hbm.at[b, :, pl.ds(lo, L_IN)], del_v)
            pltpu.sync_copy(cst_hbm.at[pl.ds(ok, 4 * CHUNK)], cst_v)

            def body(g, carry):
                base = g * 16
                rows = ldx_v[pl.ds(base, 16)]
                d = []
                for c in range(4):
                    col = jnp.full((16,), c, jnp.int32)
                    dc = plsc.load_gather(del_v, [col, rows])
                    delb[c, pl.ds(base, 16)] = dc
                    d.append(dc)
                aw = cst_v[pl.ds(base, 16)]
                ah = cst_v[pl.ds(CHUNK + base, 16)]
                acx = cst_v[pl.ds(2 * CHUNK + base, 16)]
                acy = cst_v[pl.ds(3 * CHUNK + base, 16)]
                dw = jnp.clip(d[2], -MAXR, MAXR)
                dh = jnp.clip(d[3], -MAXR, MAXR)
                pcx = acx + d[0] * aw
                pcy = acy + d[1] * ah
                pw = 0.5 * aw * jnp.exp(dw)
                ph = 0.5 * ah * jnp.exp(dh)
                r = [pcx - pw, pcy - ph, pcx + pw, pcy + ph]
                for c in range(4):
                    roib[c, pl.ds(base, 16)] = jnp.clip(r[c], -0.1, 1.1)
                return carry

            lax.fori_loop(0, GROUPS, body, 0)

            for qq, ln in ((3, LENS[3]), (0, CHUNK)):
                @pl.when((q == 3) if qq == 3 else (q < 3))
                def _(ln=ln):
                    pltpu.sync_copy(
                        delb.at[:, pl.ds(0, ln)],
                        del_out.at[b, :, pl.ds(ost, ln)])
                    pltpu.sync_copy(
                        roib.at[:, pl.ds(0, ln)],
                        roi_out.at[b, :, pl.ds(ost, ln)])

        pl.run_scoped(
            del_phase,
            pltpu.VMEM((4, L_IN), jnp.float32),
            pltpu.VMEM((4 * CHUNK,), jnp.float32),
            pltpu.VMEM((4, CHUNK), jnp.float32),
            pltpu.VMEM((4, CHUNK), jnp.float32),
        )

        def cls_phase(cls_v, clsb):
            pltpu.sync_copy(cls_hbm.at[b, :, pl.ds(lo, L_IN)], cls_v)

            def body(g, carry):
                base = g * 16
                rows = ldx_v[pl.ds(base, 16)]
                zero = jnp.full((16,), 0, jnp.int32)
                clsb[0, pl.ds(base, 16)] = plsc.load_gather(
                    cls_v, [zero, rows])
                return carry

            lax.fori_loop(0, GROUPS, body, 0)

            for qq, ln in ((3, LENS[3]), (0, CHUNK)):
                @pl.when((q == 3) if qq == 3 else (q < 3))
                def _(ln=ln):
                    pltpu.sync_copy(
                        clsb.at[:, pl.ds(0, ln)],
                        cls_out.at[b, :, pl.ds(ost, ln)])

        pl.run_scoped(
            cls_phase,
            pltpu.VMEM((1, L_IN), jnp.float32),
            pltpu.VMEM((1, CHUNK), jnp.float32),
        )

    return _roi_sc


@jax.jit
def kernel(rpn_cls, rpn_del):
    cls_t = jnp.transpose(rpn_cls, (0, 2, 1))
    del_t = jnp.transpose(rpn_del, (0, 2, 1))
    cls_o, del_o, roi_o = _build()(cls_t, del_t, _LDX, _CST)
    return (jnp.transpose(cls_o, (0, 2, 1)),
            jnp.transpose(del_o, (0, 2, 1)),
            jnp.transpose(roi_o, (0, 2, 1)))
